# baseline (speedup 1.0000x reference)
import functools

import jax
import jax.numpy as jnp
from jax import lax
from jax.experimental import pallas as pl
from jax.experimental.pallas import tpu as pltpu
from jax.experimental.pallas import tpu_sc as plsc

N = 10000
E = 320000
D = 128

NC = 2
NS = 16
NW = NC * NS
CH = 128
NCHUNK = 80
NPH = 2
PH = NCHUNK // NPH
E_PAD = NW * NCHUNK * CH
N_PAD = 10240
ROWS_PT = N_PAD // NS

def _sc_body(y_hbm, src_hbm, dst_hbm, znd_hbm, zdg_hbm, ones_hbm,
             part_hbm, degp_hbm,
             acc_sh, deg_sh, src_v, dst_v, rows_v, ones_v, sem, with_deg):
    c = lax.axis_index("c")
    s = lax.axis_index("s")
    wid = s * NC + c
    r0 = s * ROWS_PT

    pltpu.sync_copy(znd_hbm.at[pl.ds(r0, ROWS_PT)], acc_sh.at[pl.ds(r0, ROWS_PT)])
    if with_deg:
        pltpu.sync_copy(zdg_hbm.at[pl.ds(r0, ROWS_PT)], deg_sh.at[pl.ds(r0, ROWS_PT)])
        pltpu.sync_copy(ones_hbm, ones_v)
    plsc.subcore_barrier()

    def _chunk_body(i, _):
        e0 = pl.multiple_of(wid * (NCHUNK * CH) + i * CH, 8)
        pltpu.sync_copy(src_hbm.at[pl.ds(e0, CH)], src_v)
        pltpu.sync_copy(dst_hbm.at[pl.ds(e0, CH)], dst_v)
        pltpu.async_copy(y_hbm.at[src_v], rows_v, sem).wait()
        pltpu.sync_copy(rows_v, acc_sh.at[dst_v], add=True)
        if with_deg:
            pltpu.sync_copy(ones_v, deg_sh.at[dst_v], add=True)
        return 0

    lax.fori_loop(0, NCHUNK, _chunk_body, 0)
    plsc.subcore_barrier()

    o0 = c * N_PAD + r0
    pltpu.sync_copy(acc_sh.at[pl.ds(r0, ROWS_PT)], part_hbm.at[pl.ds(o0, ROWS_PT)])
    if with_deg:
        pltpu.sync_copy(deg_sh.at[pl.ds(r0, ROWS_PT)], degp_hbm.at[pl.ds(o0, ROWS_PT)])


@functools.cache
def _sc_kernels():
    mesh = plsc.VectorSubcoreMesh(core_axis_name="c", subcore_axis_name="s")

    @functools.partial(
        pl.kernel,
        out_type=[jax.ShapeDtypeStruct((2 * N_PAD, D), jnp.float32),
                  jax.ShapeDtypeStruct((2 * N_PAD,), jnp.float32)],
        mesh=mesh,
        scratch_types=[
            pltpu.VMEM_SHARED((N_PAD, D), jnp.float32),
            pltpu.VMEM_SHARED((N_PAD,), jnp.float32),
            pltpu.VMEM((CH,), jnp.int32),
            pltpu.VMEM((CH,), jnp.int32),
            pltpu.VMEM((CH, D), jnp.float32),
            pltpu.VMEM((CH,), jnp.float32),
            pltpu.SemaphoreType.DMA,
        ],
    )
    def sc_agg_deg(y_hbm, src_hbm, dst_hbm, znd_hbm, zdg_hbm, ones_hbm,
                   part_hbm, degp_hbm,
                   acc_sh, deg_sh, src_v, dst_v, rows_v, ones_v, sem):
        _sc_body(y_hbm, src_hbm, dst_hbm, znd_hbm, zdg_hbm, ones_hbm,
                 part_hbm, degp_hbm,
                 acc_sh, deg_sh, src_v, dst_v, rows_v, ones_v, sem, True)

    @functools.partial(
        pl.kernel,
        out_type=[jax.ShapeDtypeStruct((2 * N_PAD, D), jnp.float32)],
        mesh=mesh,
        scratch_types=[
            pltpu.VMEM_SHARED((N_PAD, D), jnp.float32),
            pltpu.VMEM((CH,), jnp.int32),
            pltpu.VMEM((CH,), jnp.int32),
            pltpu.VMEM((CH, D), jnp.float32),
            pltpu.SemaphoreType.DMA,
        ],
    )
    def sc_agg(y_hbm, src_hbm, dst_hbm, znd_hbm,
               part_hbm,
               acc_sh, src_v, dst_v, rows_v, sem):
        _sc_body(y_hbm, src_hbm, dst_hbm, znd_hbm, None, None, part_hbm, None,
                 acc_sh, None, src_v, dst_v, rows_v, None, sem, False)

    return sc_agg_deg, sc_agg


BN = 1000


def _mm2_body(x_ref, wm_ref, wr_ref, y_ref, r_ref):
    xb = x_ref[...]
    y_ref[...] = jnp.dot(xb, wm_ref[...], preferred_element_type=jnp.float32)
    r_ref[...] = jnp.dot(xb, wr_ref[...], preferred_element_type=jnp.float32)


_mm2 = pl.pallas_call(
    _mm2_body,
    grid=(N // BN,),
    in_specs=[pl.BlockSpec((BN, D), lambda i: (i, 0)),
              pl.BlockSpec((D, D), lambda i: (0, 0)),
              pl.BlockSpec((D, D), lambda i: (0, 0))],
    out_specs=[pl.BlockSpec((BN, D), lambda i: (i, 0)),
               pl.BlockSpec((BN, D), lambda i: (i, 0))],
    out_shape=[jax.ShapeDtypeStruct((N, D), jnp.float32),
               jax.ShapeDtypeStruct((N, D), jnp.float32)],
)


def _agg_from_partials(p_ref, dg_ref):
    deg = dg_ref[0] + dg_ref[1]
    invd = 1.0 / jnp.maximum(deg, 1.0)
    return (p_ref[0] + p_ref[1]) * invd


def _comb_body(p_ref, dg_ref, r_ref, b_ref, wm_ref, wr_ref, y2_ref, r2_ref):
    h = jnp.maximum(_agg_from_partials(p_ref, dg_ref) + r_ref[...] + b_ref[...], 0.0)
    y2_ref[...] = jnp.dot(h, wm_ref[...], preferred_element_type=jnp.float32)
    r2_ref[...] = jnp.dot(h, wr_ref[...], preferred_element_type=jnp.float32)


_comb = pl.pallas_call(
    _comb_body,
    grid=(N // BN,),
    in_specs=[pl.BlockSpec((2, BN, D), lambda i: (0, i, 0)),
              pl.BlockSpec((2, BN, 1), lambda i: (0, i, 0)),
              pl.BlockSpec((BN, D), lambda i: (i, 0)),
              pl.BlockSpec((1, D), lambda i: (0, 0)),
              pl.BlockSpec((D, D), lambda i: (0, 0)),
              pl.BlockSpec((D, D), lambda i: (0, 0))],
    out_specs=[pl.BlockSpec((BN, D), lambda i: (i, 0)),
               pl.BlockSpec((BN, D), lambda i: (i, 0))],
    out_shape=[jax.ShapeDtypeStruct((N, D), jnp.float32),
               jax.ShapeDtypeStruct((N, D), jnp.float32)],
)


def _fin_body(p_ref, dg_ref, r_ref, b_ref, o_ref):
    o_ref[...] = jnp.maximum(
        _agg_from_partials(p_ref, dg_ref) + r_ref[...] + b_ref[...], 0.0)


_fin = pl.pallas_call(
    _fin_body,
    grid=(N // BN,),
    in_specs=[pl.BlockSpec((2, BN, D), lambda i: (0, i, 0)),
              pl.BlockSpec((2, BN, 1), lambda i: (0, i, 0)),
              pl.BlockSpec((BN, D), lambda i: (i, 0)),
              pl.BlockSpec((1, D), lambda i: (0, 0))],
    out_specs=pl.BlockSpec((BN, D), lambda i: (i, 0)),
    out_shape=jax.ShapeDtypeStruct((N, D), jnp.float32),
)


def kernel(x, edge_index, W_msg1, W_root1, b1, W_msg2, W_root2, b2):
    pad = E_PAD - E
    src = jnp.concatenate([edge_index[0], jnp.zeros((pad,), jnp.int32)])
    dst = jnp.concatenate([edge_index[1], jnp.full((pad,), N_PAD - 1, jnp.int32)])
    znd = jnp.zeros((N_PAD, D), jnp.float32)
    zdg = jnp.zeros((N_PAD,), jnp.float32)
    ones = jnp.ones((CH,), jnp.float32)

    sc_agg_deg, sc_agg = _sc_kernels()
    y1, r1 = _mm2(x, W_msg1, W_root1)
    part1, degp1 = sc_agg_deg(y1, src, dst, znd, zdg, ones)
    p1 = part1.reshape(2, N_PAD, D)
    dg = degp1.reshape(2, N_PAD, 1)
    y2, r2 = _comb(p1, dg, r1, b1.reshape(1, D), W_msg2, W_root2)
    (part2,) = sc_agg(y2, src, dst, znd)
    out = _fin(part2.reshape(2, N_PAD, D), dg, r2, b2.reshape(1, D))
    return out


# speedup vs baseline: 2.1136x; 2.1136x over previous
import functools

import jax
import jax.numpy as jnp
from jax import lax
from jax.experimental import pallas as pl
from jax.experimental.pallas import tpu as pltpu
from jax.experimental.pallas import tpu_sc as plsc

N = 10000
E = 320000
D = 128

NC = 2
NS = 16
NW = NC * NS
CH = 128
NCHUNK = 80
NPH = 2
PH = NCHUNK // NPH
E_PAD = NW * NCHUNK * CH
N_PAD = 10240
ROWS_PT = N_PAD // NS

def _sc_body(y_hbm, src_hbm, dst_hbm, znd_hbm, zdg_hbm, ones_hbm,
             part_hbm, degp_hbm,
             acc_sh, deg_sh, src_v, dst_v, rows_v, ones_v, sem, with_deg):
    c = lax.axis_index("c")
    s = lax.axis_index("s")
    wid = s * NC + c
    r0 = s * ROWS_PT

    pltpu.sync_copy(znd_hbm.at[pl.ds(r0, ROWS_PT)], acc_sh.at[pl.ds(r0, ROWS_PT)])
    if with_deg:
        pltpu.sync_copy(zdg_hbm.at[pl.ds(r0, ROWS_PT)], deg_sh.at[pl.ds(r0, ROWS_PT)])
        pltpu.sync_copy(ones_hbm, ones_v)
    plsc.subcore_barrier()

    def _chunk_body(i, _):
        e0 = pl.multiple_of(wid * (NCHUNK * CH) + i * CH, 8)
        pltpu.sync_copy(src_hbm.at[pl.ds(e0, CH)], src_v)
        pltpu.sync_copy(dst_hbm.at[pl.ds(e0, CH)], dst_v)
        pltpu.async_copy(y_hbm.at[src_v], rows_v, sem).wait()
        pltpu.sync_copy(rows_v, acc_sh.at[dst_v], add=True)
        if with_deg:
            pltpu.sync_copy(ones_v, deg_sh.at[dst_v], add=True)
        return 0

    lax.fori_loop(0, NCHUNK, _chunk_body, 0)
    plsc.subcore_barrier()

    o0 = c * N_PAD + r0
    pltpu.sync_copy(acc_sh.at[pl.ds(r0, ROWS_PT)], part_hbm.at[pl.ds(o0, ROWS_PT)])
    if with_deg:
        pltpu.sync_copy(deg_sh.at[pl.ds(r0, ROWS_PT)], degp_hbm.at[pl.ds(o0, ROWS_PT)])


@functools.cache
def _sc_kernels():
    mesh = plsc.VectorSubcoreMesh(core_axis_name="c", subcore_axis_name="s")

    @functools.partial(
        pl.kernel,
        out_type=[jax.ShapeDtypeStruct((2 * N_PAD, D), jnp.float32),
                  jax.ShapeDtypeStruct((2 * N_PAD,), jnp.float32)],
        mesh=mesh,
        scratch_types=[
            pltpu.VMEM_SHARED((N_PAD, D), jnp.float32),
            pltpu.VMEM_SHARED((N_PAD,), jnp.float32),
            pltpu.VMEM((CH,), jnp.int32),
            pltpu.VMEM((CH,), jnp.int32),
            pltpu.VMEM((CH, D), jnp.float32),
            pltpu.VMEM((CH,), jnp.float32),
            pltpu.SemaphoreType.DMA,
        ],
    )
    def sc_agg_deg(y_hbm, src_hbm, dst_hbm, znd_hbm, zdg_hbm, ones_hbm,
                   part_hbm, degp_hbm,
                   acc_sh, deg_sh, src_v, dst_v, rows_v, ones_v, sem):
        _sc_body(y_hbm, src_hbm, dst_hbm, znd_hbm, zdg_hbm, ones_hbm,
                 part_hbm, degp_hbm,
                 acc_sh, deg_sh, src_v, dst_v, rows_v, ones_v, sem, True)

    @functools.partial(
        pl.kernel,
        out_type=[jax.ShapeDtypeStruct((2 * N_PAD, D), jnp.float32)],
        mesh=mesh,
        scratch_types=[
            pltpu.VMEM_SHARED((N_PAD, D), jnp.float32),
            pltpu.VMEM((CH,), jnp.int32),
            pltpu.VMEM((CH,), jnp.int32),
            pltpu.VMEM((CH, D), jnp.float32),
            pltpu.SemaphoreType.DMA,
        ],
    )
    def sc_agg(y_hbm, src_hbm, dst_hbm, znd_hbm,
               part_hbm,
               acc_sh, src_v, dst_v, rows_v, sem):
        _sc_body(y_hbm, src_hbm, dst_hbm, znd_hbm, None, None, part_hbm, None,
                 acc_sh, None, src_v, dst_v, rows_v, None, sem, False)

    return sc_agg_deg, sc_agg


BN = 1000


def _mm2_body(x_ref, wm_ref, wr_ref, y_ref, r_ref):
    xb = x_ref[...]
    y_ref[...] = jnp.dot(xb, wm_ref[...], preferred_element_type=jnp.float32)
    r_ref[...] = jnp.dot(xb, wr_ref[...], preferred_element_type=jnp.float32)


_mm2 = pl.pallas_call(
    _mm2_body,
    grid=(N // BN,),
    in_specs=[pl.BlockSpec((BN, D), lambda i: (i, 0)),
              pl.BlockSpec((D, D), lambda i: (0, 0)),
              pl.BlockSpec((D, D), lambda i: (0, 0))],
    out_specs=[pl.BlockSpec((BN, D), lambda i: (i, 0)),
               pl.BlockSpec((BN, D), lambda i: (i, 0))],
    out_shape=[jax.ShapeDtypeStruct((N, D), jnp.float32),
               jax.ShapeDtypeStruct((N, D), jnp.float32)],
)


def _agg_from_partials(p_ref, dg_ref):
    deg = dg_ref[0] + dg_ref[1]
    invd = 1.0 / jnp.maximum(deg, 1.0)
    return (p_ref[0] + p_ref[1]) * invd


def _comb_body(p_ref, dg_ref, r_ref, b_ref, wm_ref, wr_ref, y2_ref, r2_ref):
    h = jnp.maximum(_agg_from_partials(p_ref, dg_ref) + r_ref[...] + b_ref[...], 0.0)
    y2_ref[...] = jnp.dot(h, wm_ref[...], preferred_element_type=jnp.float32)
    r2_ref[...] = jnp.dot(h, wr_ref[...], preferred_element_type=jnp.float32)


_comb = pl.pallas_call(
    _comb_body,
    grid=(N // BN,),
    in_specs=[pl.BlockSpec((2, BN, D), lambda i: (0, i, 0)),
              pl.BlockSpec((2, BN, 1), lambda i: (0, i, 0)),
              pl.BlockSpec((BN, D), lambda i: (i, 0)),
              pl.BlockSpec((1, D), lambda i: (0, 0)),
              pl.BlockSpec((D, D), lambda i: (0, 0)),
              pl.BlockSpec((D, D), lambda i: (0, 0))],
    out_specs=[pl.BlockSpec((BN, D), lambda i: (i, 0)),
               pl.BlockSpec((BN, D), lambda i: (i, 0))],
    out_shape=[jax.ShapeDtypeStruct((N, D), jnp.float32),
               jax.ShapeDtypeStruct((N, D), jnp.float32)],
)


def _fin_body(p_ref, dg_ref, r_ref, b_ref, o_ref):
    o_ref[...] = jnp.maximum(
        _agg_from_partials(p_ref, dg_ref) + r_ref[...] + b_ref[...], 0.0)


_fin = pl.pallas_call(
    _fin_body,
    grid=(N // BN,),
    in_specs=[pl.BlockSpec((2, BN, D), lambda i: (0, i, 0)),
              pl.BlockSpec((2, BN, 1), lambda i: (0, i, 0)),
              pl.BlockSpec((BN, D), lambda i: (i, 0)),
              pl.BlockSpec((1, D), lambda i: (0, 0))],
    out_specs=pl.BlockSpec((BN, D), lambda i: (i, 0)),
    out_shape=jax.ShapeDtypeStruct((N, D), jnp.float32),
)


def kernel(x, edge_index, W_msg1, W_root1, b1, W_msg2, W_root2, b2):
    pad = E_PAD - E
    pad_i = jnp.arange(pad, dtype=jnp.int32)
    src = jnp.concatenate([edge_index[0], pad_i % N])
    dst = jnp.concatenate([edge_index[1], N + pad_i % (N_PAD - N)])
    znd = jnp.zeros((N_PAD, D), jnp.float32)
    zdg = jnp.zeros((N_PAD,), jnp.float32)
    ones = jnp.ones((CH,), jnp.float32)

    sc_agg_deg, sc_agg = _sc_kernels()
    y1, r1 = _mm2(x, W_msg1, W_root1)
    part1, degp1 = sc_agg_deg(y1, src, dst, znd, zdg, ones)
    p1 = part1.reshape(2, N_PAD, D)
    dg = degp1.reshape(2, N_PAD, 1)
    y2, r2 = _comb(p1, dg, r1, b1.reshape(1, D), W_msg2, W_root2)
    (part2,) = sc_agg(y2, src, dst, znd)
    out = _fin(part2.reshape(2, N_PAD, D), dg, r2, b2.reshape(1, D))
    return out


# speedup vs baseline: 4.0072x; 1.8959x over previous
import functools

import jax
import jax.numpy as jnp
from jax import lax
from jax.experimental import pallas as pl
from jax.experimental.pallas import tpu as pltpu
from jax.experimental.pallas import tpu_sc as plsc

N = 10000
E = 320000
D = 128

NC = 2
NS = 16
NW = NC * NS
CH = 128
NCHUNK = 80
NPH = 2
PH = NCHUNK // NPH
E_PAD = NW * NCHUNK * CH
N_PAD = 10240
ROWS_PT = N_PAD // NS

def _sc_body(y_hbm, src_hbm, dst_hbm, znd_hbm, zdg_hbm, ones_hbm,
             part_hbm, degp_hbm,
             acc_sh, deg_sh, src_all, dst_all, rows0, rows1, ones_v,
             sem0, sem1, with_deg):
    c = lax.axis_index("c")
    s = lax.axis_index("s")
    wid = s * NC + c
    r0 = s * ROWS_PT

    rows = [rows0, rows1]
    sems = [sem0, sem1]
    c0 = pl.multiple_of(wid * NCHUNK, 8)

    pltpu.sync_copy(znd_hbm.at[pl.ds(r0, ROWS_PT)], acc_sh.at[pl.ds(r0, ROWS_PT)])
    if with_deg:
        pltpu.sync_copy(zdg_hbm.at[pl.ds(r0, ROWS_PT)], deg_sh.at[pl.ds(r0, ROWS_PT)])
        pltpu.sync_copy(ones_hbm, ones_v)
    plsc.subcore_barrier()

    for p in range(NPH):
        pltpu.sync_copy(src_hbm.at[pl.ds(c0 + p * PH, PH)], src_all)
        pltpu.sync_copy(dst_hbm.at[pl.ds(c0 + p * PH, PH)], dst_all)
        pltpu.async_copy(y_hbm.at[src_all.at[0]], rows[0], sems[0])
        pltpu.async_copy(y_hbm.at[src_all.at[1]], rows[1], sems[1])

        def _pair_body(j, _):
            for b in range(2):
                i = j * 2 + b
                pltpu.make_async_copy(
                    y_hbm.at[src_all.at[i]], rows[b], sems[b]).wait()
                pltpu.sync_copy(rows[b], acc_sh.at[dst_all.at[i]], add=True)
                if with_deg:
                    pltpu.sync_copy(ones_v, deg_sh.at[dst_all.at[i]], add=True)

                @pl.when(i + 2 < PH)
                def _():
                    pltpu.async_copy(y_hbm.at[src_all.at[i + 2]], rows[b], sems[b])
            return 0

        lax.fori_loop(0, PH // 2, _pair_body, 0)
    plsc.subcore_barrier()

    o0 = c * N_PAD + r0
    pltpu.sync_copy(acc_sh.at[pl.ds(r0, ROWS_PT)], part_hbm.at[pl.ds(o0, ROWS_PT)])
    if with_deg:
        pltpu.sync_copy(deg_sh.at[pl.ds(r0, ROWS_PT)], degp_hbm.at[pl.ds(o0, ROWS_PT)])


@functools.cache
def _sc_kernels():
    mesh = plsc.VectorSubcoreMesh(core_axis_name="c", subcore_axis_name="s")

    @functools.partial(
        pl.kernel,
        out_type=[jax.ShapeDtypeStruct((2 * N_PAD, D), jnp.float32),
                  jax.ShapeDtypeStruct((2 * N_PAD,), jnp.float32)],
        mesh=mesh,
        scratch_types=[
            pltpu.VMEM_SHARED((N_PAD, D), jnp.float32),
            pltpu.VMEM_SHARED((N_PAD,), jnp.float32),
            pltpu.VMEM((PH, CH), jnp.int32),
            pltpu.VMEM((PH, CH), jnp.int32),
            pltpu.VMEM((CH, D), jnp.float32),
            pltpu.VMEM((CH, D), jnp.float32),
            pltpu.VMEM((CH,), jnp.float32),
            pltpu.SemaphoreType.DMA,
            pltpu.SemaphoreType.DMA,
        ],
    )
    def sc_agg_deg(y_hbm, src_hbm, dst_hbm, znd_hbm, zdg_hbm, ones_hbm,
                   part_hbm, degp_hbm,
                   acc_sh, deg_sh, src_all, dst_all, rows0, rows1, ones_v,
                   sem0, sem1):
        _sc_body(y_hbm, src_hbm, dst_hbm, znd_hbm, zdg_hbm, ones_hbm,
                 part_hbm, degp_hbm,
                 acc_sh, deg_sh, src_all, dst_all, rows0, rows1, ones_v,
                 sem0, sem1, True)

    @functools.partial(
        pl.kernel,
        out_type=[jax.ShapeDtypeStruct((2 * N_PAD, D), jnp.float32)],
        mesh=mesh,
        scratch_types=[
            pltpu.VMEM_SHARED((N_PAD, D), jnp.float32),
            pltpu.VMEM((PH, CH), jnp.int32),
            pltpu.VMEM((PH, CH), jnp.int32),
            pltpu.VMEM((CH, D), jnp.float32),
            pltpu.VMEM((CH, D), jnp.float32),
            pltpu.SemaphoreType.DMA,
            pltpu.SemaphoreType.DMA,
        ],
    )
    def sc_agg(y_hbm, src_hbm, dst_hbm, znd_hbm,
               part_hbm,
               acc_sh, src_all, dst_all, rows0, rows1, sem0, sem1):
        _sc_body(y_hbm, src_hbm, dst_hbm, znd_hbm, None, None, part_hbm, None,
                 acc_sh, None, src_all, dst_all, rows0, rows1, None,
                 sem0, sem1, False)

    return sc_agg_deg, sc_agg


BN = 1000


def _mm2_body(x_ref, wm_ref, wr_ref, y_ref, r_ref):
    xb = x_ref[...]
    y_ref[...] = jnp.dot(xb, wm_ref[...], preferred_element_type=jnp.float32)
    r_ref[...] = jnp.dot(xb, wr_ref[...], preferred_element_type=jnp.float32)


_mm2 = pl.pallas_call(
    _mm2_body,
    grid=(N // BN,),
    in_specs=[pl.BlockSpec((BN, D), lambda i: (i, 0)),
              pl.BlockSpec((D, D), lambda i: (0, 0)),
              pl.BlockSpec((D, D), lambda i: (0, 0))],
    out_specs=[pl.BlockSpec((BN, D), lambda i: (i, 0)),
               pl.BlockSpec((BN, D), lambda i: (i, 0))],
    out_shape=[jax.ShapeDtypeStruct((N, D), jnp.float32),
               jax.ShapeDtypeStruct((N, D), jnp.float32)],
)


def _agg_from_partials(p_ref, dg_ref):
    deg = dg_ref[0] + dg_ref[1]
    invd = 1.0 / jnp.maximum(deg, 1.0)
    return (p_ref[0] + p_ref[1]) * invd


def _comb_body(p_ref, dg_ref, r_ref, b_ref, wm_ref, wr_ref, y2_ref, r2_ref):
    h = jnp.maximum(_agg_from_partials(p_ref, dg_ref) + r_ref[...] + b_ref[...], 0.0)
    y2_ref[...] = jnp.dot(h, wm_ref[...], preferred_element_type=jnp.float32)
    r2_ref[...] = jnp.dot(h, wr_ref[...], preferred_element_type=jnp.float32)


_comb = pl.pallas_call(
    _comb_body,
    grid=(N // BN,),
    in_specs=[pl.BlockSpec((2, BN, D), lambda i: (0, i, 0)),
              pl.BlockSpec((2, BN, 1), lambda i: (0, i, 0)),
              pl.BlockSpec((BN, D), lambda i: (i, 0)),
              pl.BlockSpec((1, D), lambda i: (0, 0)),
              pl.BlockSpec((D, D), lambda i: (0, 0)),
              pl.BlockSpec((D, D), lambda i: (0, 0))],
    out_specs=[pl.BlockSpec((BN, D), lambda i: (i, 0)),
               pl.BlockSpec((BN, D), lambda i: (i, 0))],
    out_shape=[jax.ShapeDtypeStruct((N, D), jnp.float32),
               jax.ShapeDtypeStruct((N, D), jnp.float32)],
)


def _fin_body(p_ref, dg_ref, r_ref, b_ref, o_ref):
    o_ref[...] = jnp.maximum(
        _agg_from_partials(p_ref, dg_ref) + r_ref[...] + b_ref[...], 0.0)


_fin = pl.pallas_call(
    _fin_body,
    grid=(N // BN,),
    in_specs=[pl.BlockSpec((2, BN, D), lambda i: (0, i, 0)),
              pl.BlockSpec((2, BN, 1), lambda i: (0, i, 0)),
              pl.BlockSpec((BN, D), lambda i: (i, 0)),
              pl.BlockSpec((1, D), lambda i: (0, 0))],
    out_specs=pl.BlockSpec((BN, D), lambda i: (i, 0)),
    out_shape=jax.ShapeDtypeStruct((N, D), jnp.float32),
)


def kernel(x, edge_index, W_msg1, W_root1, b1, W_msg2, W_root2, b2):
    pad = E_PAD - E
    pad_i = jnp.arange(pad, dtype=jnp.int32)
    src = jnp.concatenate([edge_index[0], pad_i % N]).reshape(NW * NCHUNK, CH)
    dst = jnp.concatenate(
        [edge_index[1], N + pad_i % (N_PAD - N)]).reshape(NW * NCHUNK, CH)
    znd = jnp.zeros((N_PAD, D), jnp.float32)
    zdg = jnp.zeros((N_PAD,), jnp.float32)
    ones = jnp.ones((CH,), jnp.float32)

    sc_agg_deg, sc_agg = _sc_kernels()
    y1, r1 = _mm2(x, W_msg1, W_root1)
    part1, degp1 = sc_agg_deg(y1, src, dst, znd, zdg, ones)
    p1 = part1.reshape(2, N_PAD, D)
    dg = degp1.reshape(2, N_PAD, 1)
    y2, r2 = _comb(p1, dg, r1, b1.reshape(1, D), W_msg2, W_root2)
    (part2,) = sc_agg(y2, src, dst, znd)
    out = _fin(part2.reshape(2, N_PAD, D), dg, r2, b2.reshape(1, D))
    return out
